# baseline (speedup 1.0000x reference)
import functools

import jax
import jax.numpy as jnp
from jax import lax
from jax.experimental import pallas as pl
from jax.experimental.pallas import tpu as pltpu
from jax.experimental.pallas import tpu_sc as plsc

VOCAB = 1000
EMB_DIM = 128
HID = 8
HID_P = 16
VPAD = 1024
PACK = 128 // HID_P
NC = 2
NS = 16
NW = NC * NS
LANES = 16


def _table_body(embp_ref, wa_ref, wb_ref, t_ref):
    dn = (((1,), (1,)), ((), ()))
    t_ref[0:VPAD, :] = lax.dot_general(
        embp_ref[...], wa_ref[...], dn,
        preferred_element_type=jnp.float32)[:, 0:HID_P]
    t_ref[VPAD:2 * VPAD, :] = lax.dot_general(
        embp_ref[...], wb_ref[...], dn,
        preferred_element_type=jnp.float32)[:, 0:HID_P]


def _build_table(embp, wa, wb):
    return pl.pallas_call(
        _table_body,
        out_shape=jax.ShapeDtypeStruct((2 * VPAD, HID_P), jnp.float32),
    )(embp, wa, wb)


def _sc_gather(table_packed, idx0, idx1, batch):
    bpw = batch // NW
    ngrp = bpw // LANES
    trows = 2 * VPAD // PACK
    mesh = plsc.VectorSubcoreMesh(core_axis_name="c", subcore_axis_name="s")

    @functools.partial(
        pl.kernel, mesh=mesh,
        compiler_params=pltpu.CompilerParams(needs_layout_passes=False),
        out_type=jax.ShapeDtypeStruct((NW, HID_P, bpw), jnp.float32),
        scratch_types=[
            pltpu.VMEM((trows * 128,), jnp.float32),
            pltpu.VMEM((bpw,), jnp.int32),
            pltpu.VMEM((bpw,), jnp.int32),
            pltpu.VMEM((HID_P, bpw), jnp.float32),
        ],
    )
    def gather_k(table_hbm, idx0_hbm, idx1_hbm, out_hbm,
                 tab_v, i0_v, i1_v, ht_v):
        wid = lax.axis_index("s") * NC + lax.axis_index("c")
        base = wid * bpw
        pltpu.sync_copy(table_hbm, tab_v)
        pltpu.sync_copy(idx0_hbm.at[pl.ds(base, bpw)], i0_v)
        pltpu.sync_copy(idx1_hbm.at[pl.ds(base, bpw)], i1_v)

        for g in range(ngrp):
            iv0 = i0_v[pl.ds(g * LANES, LANES)]
            iv1 = i1_v[pl.ds(g * LANES, LANES)]
            f0 = lax.shift_left(iv0, 4)
            f1 = lax.shift_left(iv1, 4)
            for c in range(HID_P):
                v0 = plsc.load_gather(tab_v, [f0 + c])
                v1 = plsc.load_gather(tab_v, [f1 + c])
                ht_v[c, pl.ds(g * LANES, LANES)] = v0 + v1
        pltpu.sync_copy(ht_v, out_hbm.at[wid])

    return gather_k(table_packed, idx0, idx1)


def _mlp_body(h_ref, w2_ref, b1_ref, b2_ref, out_ref):
    ht = jnp.tanh(h_ref[0] + b1_ref[...])
    dn = (((0,), (1,)), ((), ()))
    acc = lax.dot_general(ht, w2_ref[...], dn, preferred_element_type=jnp.float32)
    out_ref[...] = acc + b2_ref[...]


def kernel(x, emb, fc1_w, fc1_b, fc2_w, fc2_b):
    x = x.astype(jnp.int32)
    batch = x.shape[0]
    bpw = batch // NW

    embp = jnp.pad(emb, ((0, VPAD - VOCAB), (0, 0)))
    w1p = jnp.pad(fc1_w, ((0, HID_P - HID), (0, 0)))
    table = _build_table(embp, w1p[:, :EMB_DIM], w1p[:, EMB_DIM:])
    table_packed = table.reshape(2 * VPAD * HID_P)

    h = _sc_gather(table_packed, x[:, 0], x[:, 1] + VPAD, batch)

    w2p = jnp.pad(fc2_w, ((0, 0), (0, HID_P - HID)))
    b1p = jnp.pad(fc1_b, (0, HID_P - HID)).reshape(HID_P, 1)
    b2 = fc2_b.reshape(1, VOCAB)
    return pl.pallas_call(
        _mlp_body,
        grid=(NW,),
        in_specs=[
            pl.BlockSpec((1, HID_P, bpw), lambda i: (i, 0, 0)),
            pl.BlockSpec((VOCAB, HID_P), lambda i: (0, 0)),
            pl.BlockSpec((HID_P, 1), lambda i: (0, 0)),
            pl.BlockSpec((1, VOCAB), lambda i: (0, 0)),
        ],
        out_specs=pl.BlockSpec((bpw, VOCAB), lambda i: (i, 0)),
        out_shape=jax.ShapeDtypeStruct((batch, VOCAB), jnp.float32),
    )(h, w2p, b1p, b2)
